# Initial kernel scaffold; baseline (speedup 1.0000x reference)
import jax
import jax.numpy as jnp
from jax.experimental import pallas as pl


def kernel(x, query, metric, token_size):
    raise NotImplementedError("write your pallas kernel here")


# speedup vs baseline: 239.8060x; 239.8060x over previous
import functools

import jax
import jax.numpy as jnp
from jax.experimental import pallas as pl
from jax.experimental.pallas import tpu as pltpu

_N, _T, _C = 32, 577, 768
_R = 256
_K = _T - _R

_NEG = float("-inf")


def _to_row(v_col, iota_r, iota_c):
    return jnp.sum(jnp.where(iota_r == iota_c, v_col, 0.0), axis=0,
                   keepdims=True)


def _body(x_ref, q_ref, m_ref, ts_ref, xo_ref, tso_ref):
    t, c, r, k = _T, _C, _R, _K
    f32 = jnp.float32

    m = m_ref[0]
    q = q_ref[0]
    xr = x_ref[0]
    ts_c = ts_ref[0]

    iota_r = jax.lax.broadcasted_iota(jnp.int32, (t, t), 0).astype(f32)
    iota_c = jax.lax.broadcasted_iota(jnp.int32, (t, t), 1).astype(f32)

    norm = jnp.sqrt(jnp.sum(m * m, axis=-1, keepdims=True))
    mn = m / norm
    sim = jax.lax.dot_general(
        mn, mn, (((1,), (1,)), ((), ())),
        preferred_element_type=f32, precision=jax.lax.Precision.DEFAULT)
    protect = ((iota_r == 0.0) | (iota_r == t - 1.0)
               | (iota_c == 0.0) | (iota_c == t - 1.0) | (iota_r == iota_c))
    sim = jnp.where(protect, _NEG, sim)

    nm_c = jnp.max(sim, axis=1, keepdims=True)
    nm_r = jnp.max(sim, axis=0, keepdims=True)
    allowed = (nm_r < nm_c) | ((nm_r == nm_c) & (iota_c > iota_r))
    mm_c = jnp.max(jnp.where(allowed, sim, _NEG), axis=1, keepdims=True)
    allowed_t = (nm_c < nm_r) | ((nm_c == nm_r) & (iota_r > iota_c))
    mm_r = jnp.max(jnp.where(allowed_t, sim, _NEG), axis=0, keepdims=True)

    q_last = q[:, c - 1:c]
    imp_c = (jnp.sum(q, axis=-1, keepdims=True) - q_last) / (c - 1) + q_last
    imp_r = _to_row(imp_c, iota_r, iota_c)
    score_c = imp_c - mm_c
    score_r = imp_r - mm_r

    cmp = (score_r < score_c) | ((score_r == score_c) & (iota_c < iota_r))
    rank_c = jnp.sum(cmp.astype(f32), axis=1, keepdims=True)
    cmp_t = (score_c < score_r) | ((score_c == score_r) & (iota_r < iota_c))
    rank_r = jnp.sum(cmp_t.astype(f32), axis=0, keepdims=True)
    is_src_c = rank_c < r
    is_dst_c = ~is_src_c
    is_src_r = rank_r < r
    is_dst_r = ~is_src_r

    pos_r = jnp.sum((is_dst_c & (iota_r < iota_c)).astype(f32), axis=0,
                    keepdims=True)

    simd = jnp.where(is_dst_r, sim, _NEG)
    cm_c = jnp.max(simd, axis=1, keepdims=True)
    choice_c = jnp.min(jnp.where(simd == cm_c, iota_c, f32(t)), axis=1,
                       keepdims=True)
    cmat = iota_c == choice_c

    cpos_c = jnp.sum(jnp.where(cmat, pos_r, 0.0), axis=1, keepdims=True)
    ichoice_c = jnp.sum(jnp.where(cmat, imp_r, 0.0), axis=1, keepdims=True)

    mx = jnp.maximum(imp_c, ichoice_c)
    es = jnp.exp(imp_c - mx)
    ed = jnp.exp(ichoice_c - mx)
    w0 = 2.0 * es / (es + ed)
    srcf_c = is_src_c.astype(f32)
    aw_c = srcf_c * w0
    bw_c = srcf_c * (1.0 - w0)
    tsrc_c = srcf_c * ts_c

    coefdiag_r = 1.0 + jnp.sum(jnp.where(cmat, bw_c, 0.0), axis=0,
                               keepdims=True)
    ts_r = _to_row(ts_c, iota_r, iota_c)
    tst_r = ts_r + jnp.sum(jnp.where(cmat, tsrc_c, 0.0), axis=0,
                           keepdims=True)
    cpos_r = _to_row(cpos_c, iota_r, iota_c)
    aw_r = _to_row(aw_c, iota_r, iota_c)

    iota_k = jax.lax.broadcasted_iota(jnp.int32, (k, t), 0).astype(f32)
    g = (pos_r == iota_k) & is_dst_r
    src_hit = (cpos_r == iota_k) & is_src_r
    mmat = (jnp.where(g, coefdiag_r, 0.0) + jnp.where(src_hit, aw_r, 0.0))
    tstc = jnp.sum(jnp.where(g, tst_r, 0.0), axis=1, keepdims=True)
    mmat = mmat / tstc

    xs = xr * ts_c
    out = jax.lax.dot_general(
        mmat, xs, (((1,), (0,)), ((), ())),
        preferred_element_type=f32, precision=jax.lax.Precision.HIGHEST)
    xo_ref[0] = out
    tso_ref[0] = tstc


@jax.jit
def kernel(x, query, metric, token_size):
    n, t, c, k = _N, _T, _C, _K
    grid = (n,)
    out = pl.pallas_call(
        _body,
        grid=grid,
        in_specs=[
            pl.BlockSpec((1, t, c), lambda b: (b, 0, 0)),
            pl.BlockSpec((1, t, c), lambda b: (b, 0, 0)),
            pl.BlockSpec((1, t, c), lambda b: (b, 0, 0)),
            pl.BlockSpec((1, t, 1), lambda b: (b, 0, 0)),
        ],
        out_specs=[
            pl.BlockSpec((1, k, c), lambda b: (b, 0, 0)),
            pl.BlockSpec((1, k, 1), lambda b: (b, 0, 0)),
        ],
        out_shape=[
            jax.ShapeDtypeStruct((n, k, c), jnp.float32),
            jax.ShapeDtypeStruct((n, k, 1), jnp.float32),
        ],
        compiler_params=pltpu.CompilerParams(
            dimension_semantics=("arbitrary",)),
    )(x, query, metric, token_size)
    return (out[0], out[1])


# speedup vs baseline: 261.9950x; 1.0925x over previous
import functools

import jax
import jax.numpy as jnp
from jax.experimental import pallas as pl
from jax.experimental.pallas import tpu as pltpu

_N, _T, _C = 32, 577, 768
_R = 256
_K = _T - _R

_NEG = float("-inf")


def _body(x_ref, q_ref, m_ref, ts_ref, xo_ref, tso_ref):
    t, c, r, k = _T, _C, _R, _K
    f32 = jnp.float32

    m = m_ref[0]
    q = q_ref[0]
    xr = x_ref[0]
    ts_c = ts_ref[0]

    iota_r = jax.lax.broadcasted_iota(jnp.int32, (t, t), 0).astype(f32)
    iota_c = jax.lax.broadcasted_iota(jnp.int32, (t, t), 1).astype(f32)

    norm = jnp.sqrt(jnp.sum(m * m, axis=-1, keepdims=True))
    mn = m / norm
    sim = jax.lax.dot_general(
        mn, mn, (((1,), (1,)), ((), ())),
        preferred_element_type=f32, precision=jax.lax.Precision.DEFAULT)
    protect = ((iota_r == 0.0) | (iota_r == t - 1.0)
               | (iota_c == 0.0) | (iota_c == t - 1.0) | (iota_r == iota_c))
    sim = jnp.where(protect, _NEG, sim)

    nm_c = jnp.max(sim, axis=1, keepdims=True)
    nm_r = jnp.transpose(nm_c)
    allowed = (nm_r < nm_c) | ((nm_r == nm_c) & (iota_c > iota_r))
    mm_c = jnp.max(jnp.where(allowed, sim, _NEG), axis=1, keepdims=True)

    q_last = q[:, c - 1:c]
    imp_c = (jnp.sum(q, axis=-1, keepdims=True) - q_last) / (c - 1) + q_last
    score_c = imp_c - mm_c
    score_r = jnp.transpose(score_c)
    imp_ts = jnp.transpose(jnp.concatenate([imp_c, ts_c], axis=1))
    imp_r = imp_ts[0:1, :]
    ts_r = imp_ts[1:2, :]

    cmp = (score_r < score_c) | ((score_r == score_c) & (iota_c < iota_r))
    rank_c = jnp.sum(cmp.astype(f32), axis=1, keepdims=True)
    rank_r = jnp.transpose(rank_c)
    is_src_c = rank_c < r
    is_dst_c = ~is_src_c
    is_src_r = rank_r < r
    is_dst_r = ~is_src_r

    pos_r = jnp.sum((is_dst_c & (iota_r < iota_c)).astype(f32), axis=0,
                    keepdims=True)

    simd = jnp.where(is_dst_r, sim, _NEG)
    cm_c = jnp.max(simd, axis=1, keepdims=True)
    choice_c = jnp.min(jnp.where(simd == cm_c, iota_c, f32(t)), axis=1,
                       keepdims=True)
    cmat = iota_c == choice_c

    cpos_c = jnp.sum(jnp.where(cmat, pos_r, 0.0), axis=1, keepdims=True)
    ichoice_c = jnp.sum(jnp.where(cmat, imp_r, 0.0), axis=1, keepdims=True)

    mx = jnp.maximum(imp_c, ichoice_c)
    es = jnp.exp(imp_c - mx)
    ed = jnp.exp(ichoice_c - mx)
    w0 = 2.0 * es / (es + ed)
    srcf_c = is_src_c.astype(f32)
    aw_c = srcf_c * w0
    bw_c = srcf_c * (1.0 - w0)
    tsrc_c = srcf_c * ts_c

    coefdiag_r = 1.0 + jnp.sum(jnp.where(cmat, bw_c, 0.0), axis=0,
                               keepdims=True)
    tst_r = ts_r + jnp.sum(jnp.where(cmat, tsrc_c, 0.0), axis=0,
                           keepdims=True)
    cpos_aw = jnp.transpose(jnp.concatenate([cpos_c, aw_c], axis=1))
    cpos_r = cpos_aw[0:1, :]
    aw_r = cpos_aw[1:2, :]

    iota_k = jax.lax.broadcasted_iota(jnp.int32, (k, t), 0).astype(f32)
    g = (pos_r == iota_k) & is_dst_r
    src_hit = (cpos_r == iota_k) & is_src_r
    mmat = (jnp.where(g, coefdiag_r, 0.0) + jnp.where(src_hit, aw_r, 0.0))
    tstc = jnp.sum(jnp.where(g, tst_r, 0.0), axis=1, keepdims=True)
    mmat = mmat / tstc

    xs = xr * ts_c
    out = jax.lax.dot_general(
        mmat, xs, (((1,), (0,)), ((), ())),
        preferred_element_type=f32, precision=jax.lax.Precision.HIGHEST)
    xo_ref[0] = out
    tso_ref[0] = tstc


@jax.jit
def kernel(x, query, metric, token_size):
    n, t, c, k = _N, _T, _C, _K
    grid = (n,)
    out = pl.pallas_call(
        _body,
        grid=grid,
        in_specs=[
            pl.BlockSpec((1, t, c), lambda b: (b, 0, 0)),
            pl.BlockSpec((1, t, c), lambda b: (b, 0, 0)),
            pl.BlockSpec((1, t, c), lambda b: (b, 0, 0)),
            pl.BlockSpec((1, t, 1), lambda b: (b, 0, 0)),
        ],
        out_specs=[
            pl.BlockSpec((1, k, c), lambda b: (b, 0, 0)),
            pl.BlockSpec((1, k, 1), lambda b: (b, 0, 0)),
        ],
        out_shape=[
            jax.ShapeDtypeStruct((n, k, c), jnp.float32),
            jax.ShapeDtypeStruct((n, k, 1), jnp.float32),
        ],
        compiler_params=pltpu.CompilerParams(
            dimension_semantics=("arbitrary",)),
    )(x, query, metric, token_size)
    return (out[0], out[1])
